# baseline (speedup 1.0000x reference)
import functools

import jax
import jax.numpy as jnp
from jax import lax
from jax.experimental import pallas as pl
from jax.experimental.pallas import tpu as pltpu
from jax.experimental.pallas import tpu_sc as plsc

N_NODES = 10000
N_EDGES = 320000
DIM = 128
HALF = DIM // 2
NPAD = 10240
E_TILE = N_EDGES // 16
BLK = 80
SUP = 2000
N_SUP = E_TILE // SUP
G_PER_SUP = SUP // BLK
SLICE = NPAD // 16
ROWS_T = NPAD // 16


def _tc_front_body(f_ref, w_ref, a_ref, z0_ref, z1_ref, s_ref):
    z = jnp.dot(f_ref[...], w_ref[...], preferred_element_type=jnp.float32)
    z0_ref[...] = z[:, :HALF]
    z1_ref[...] = z[:, HALF:]
    s_ref[...] = jnp.dot(z, a_ref[...], preferred_element_type=jnp.float32)


def _tc_front(feature, fc_weight, attn2):
    blk = 2000
    grid = N_NODES // blk
    return pl.pallas_call(
        _tc_front_body,
        grid=(grid,),
        in_specs=[
            pl.BlockSpec((blk, DIM), lambda i: (i, 0)),
            pl.BlockSpec((DIM, DIM), lambda i: (0, 0)),
            pl.BlockSpec((DIM, 2), lambda i: (0, 0)),
        ],
        out_specs=[
            pl.BlockSpec((blk, HALF), lambda i: (i, 0)),
            pl.BlockSpec((blk, HALF), lambda i: (i, 0)),
            pl.BlockSpec((blk, 2), lambda i: (i, 0)),
        ],
        out_shape=[
            jax.ShapeDtypeStruct((N_NODES, HALF), jnp.float32),
            jax.ShapeDtypeStruct((N_NODES, HALF), jnp.float32),
            jax.ShapeDtypeStruct((N_NODES, 2), jnp.float32),
        ],
    )(feature, fc_weight, attn2)


def _tc_norm_body(u0_ref, u1_ref, d_ref, o_ref):
    d = d_ref[...]
    d = jnp.where(d == 0.0, 1.0, d)
    o_ref[...] = jnp.concatenate([u0_ref[...] / d, u1_ref[...] / d], axis=1)


def _tc_norm(u0, u1, den):
    blk = 2000
    grid = N_NODES // blk
    return pl.pallas_call(
        _tc_norm_body,
        grid=(grid,),
        in_specs=[
            pl.BlockSpec((blk, HALF), lambda i: (i, 0)),
            pl.BlockSpec((blk, HALF), lambda i: (i, 0)),
            pl.BlockSpec((blk, 1), lambda i: (i, 0)),
        ],
        out_specs=pl.BlockSpec((blk, DIM), lambda i: (i, 0)),
        out_shape=jax.ShapeDtypeStruct((N_NODES, DIM), jnp.float32),
    )(u0, u1, den)


def _sc_gat(z0, z1, s1, s2, src, dst):
    mesh = plsc.VectorSubcoreMesh(core_axis_name="c", subcore_axis_name="s")

    @functools.partial(
        pl.kernel,
        mesh=mesh,
        compiler_params=pltpu.CompilerParams(needs_layout_passes=False, use_tc_tiling_on_sc=False),
        out_type=[
            jax.ShapeDtypeStruct((2, NPAD, HALF), jnp.float32),
            jax.ShapeDtypeStruct((2, 1, NPAD), jnp.float32),
        ],
        scratch_types=[
            pltpu.VMEM((N_NODES,), jnp.float32),
            pltpu.VMEM((N_NODES,), jnp.float32),
            pltpu.VMEM((NPAD,), jnp.float32),
            pltpu.VMEM((NPAD,), jnp.float32),
            pltpu.VMEM((2 * SUP,), jnp.int32),
            pltpu.VMEM((2 * SUP,), jnp.int32),
            pltpu.VMEM((SUP,), jnp.float32),
            pltpu.VMEM((BLK, HALF), jnp.float32),
            pltpu.VMEM((BLK, HALF), jnp.float32),
            pltpu.VMEM((BLK, HALF), jnp.float32),
            pltpu.VMEM((3, BLK), jnp.int32),
            pltpu.VMEM((SLICE,), jnp.float32),
            pltpu.VMEM((16, SLICE), jnp.float32),
            pltpu.VMEM_SHARED((16, 1, NPAD), jnp.float32),
            pltpu.VMEM_SHARED((NPAD,), jnp.float32),
            pltpu.VMEM_SHARED((NPAD, HALF), jnp.float32),
            pltpu.SemaphoreType.DMA,
            pltpu.SemaphoreType.DMA,
            pltpu.SemaphoreType.DMA,
            pltpu.SemaphoreType.DMA,
            pltpu.SemaphoreType.DMA,
            pltpu.SemaphoreType.DMA,
            pltpu.SemaphoreType.DMA,
            pltpu.SemaphoreType.DMA,
            pltpu.SemaphoreType.DMA,
        ],
    )
    def k(z_hbm0, z_hbm1, s1_hbm, s2_hbm, src_hbm, dst_hbm, u_out, den_out,
          s1_v, s2_v, m_v, den_v, sbuf, dbuf, ee_v, zbuf0, zbuf1, zbuf2,
          idx_d, acc_v, comb2, red_sh, mfull_sh, h_sh,
          gsem0, gsem1, gsem2, ssem0, ssem1, ssem2, isem_s, isem_d, zsem):
        c = lax.axis_index("c")
        s = lax.axis_index("s")
        ebase = s * E_TILE

        scope = jax.named_scope
        pltpu.async_copy(s1_hbm, s1_v, isem_s)
        pltpu.async_copy(s2_hbm, s2_v, isem_d)

        zf = jnp.zeros((16,), jnp.float32)

        def zinit(i, _):
            m_v[pl.ds(i * 16, 16)] = zf
            den_v[pl.ds(i * 16, 16)] = zf
            return 0

        lax.fori_loop(0, NPAD // 16, zinit, 0)

        def zrow(r, _):
            for kk in range(HALF // 16):
                zbuf0[r, pl.ds(kk * 16, 16)] = zf
            return 0

        lax.fori_loop(0, BLK, zrow, 0)

        for g in range(ROWS_T // BLK):
            pltpu.async_copy(zbuf0, h_sh.at[pl.ds(s * ROWS_T + g * BLK, BLK)],
                             zsem)

        pltpu.make_async_copy(s1_hbm, s1_v, isem_s).wait()
        pltpu.make_async_copy(s2_hbm, s2_v, isem_d).wait()

        def start_idx(b, off):
            base = ebase + b * SUP
            pltpu.async_copy(src_hbm.at[pl.ds(base, SUP)],
                             sbuf.at[pl.ds(off, SUP)], isem_s)
            pltpu.async_copy(dst_hbm.at[pl.ds(base, SUP)],
                             dbuf.at[pl.ds(off, SUP)], isem_d)

        def drain_idx():
            pltpu.make_async_copy(src_hbm.at[pl.ds(0, SUP)],
                                  sbuf.at[pl.ds(0, SUP)], isem_s).wait()
            pltpu.make_async_copy(dst_hbm.at[pl.ds(0, SUP)],
                                  dbuf.at[pl.ds(0, SUP)], isem_d).wait()

        start_idx(0, 0)

        def scatter_max(didx, e):
            def cond(active):
                return active > 0

            def body(active):
                cur = plsc.load_gather(m_v, [didx])
                pend = cur < e
                plsc.store_scatter(m_v, [didx], jnp.maximum(cur, e), mask=pend)
                chk = plsc.load_gather(m_v, [didx])
                return jnp.any(chk < e).astype(jnp.int32)

            lax.while_loop(cond, body, jnp.int32(1))

        def phase_a(b, _):
            off = lax.rem(b, 2) * SUP
            drain_idx()

            @pl.when(b + 1 < N_SUP)
            def _():
                start_idx(b + 1, SUP - off)

            def vec(j, _):
                sl = pl.ds(off + j * 16, 16)
                sidx = sbuf[sl]
                didx = dbuf[sl]
                e = jnp.maximum(plsc.load_gather(s1_v, [sidx])
                                + plsc.load_gather(s2_v, [didx]), 0.0)
                scatter_max(didx, e)
                return 0

            lax.fori_loop(0, SUP // 16, vec, 0)
            return 0

        with scope("phaseA"):
            lax.fori_loop(0, N_SUP, phase_a, 0)

        for g in range(ROWS_T // BLK):
            pltpu.make_async_copy(zbuf0, h_sh.at[pl.ds(0, BLK)], zsem).wait()

        def combine(local_v, op):
            plsc.subcore_barrier()
            pltpu.sync_copy(local_v, red_sh.at[s, 0])
            plsc.subcore_barrier()
            pltpu.sync_copy(red_sh.at[:, 0, pl.ds(s * SLICE, SLICE)], comb2)

            def red(j, _):
                sl = pl.ds(j * 16, 16)
                a = comb2[0, sl]
                for i in range(1, 16):
                    a = op(a, comb2[i, sl])
                acc_v[sl] = a
                return 0

            lax.fori_loop(0, SLICE // 16, red, 0)

        with scope("combineM"):
            combine(m_v, jnp.maximum)
            pltpu.sync_copy(acc_v, mfull_sh.at[pl.ds(s * SLICE, SLICE)])
        plsc.subcore_barrier()
        pltpu.sync_copy(mfull_sh, m_v)

        bufs = (zbuf0, zbuf1, zbuf2)
        gsems = (gsem0, gsem1, gsem2)
        ssems = (ssem0, ssem1, ssem2)

        def drain(buf, sem):
            pltpu.make_async_copy(z_hbm0.at[pl.ds(0, BLK)], buf, sem).wait()

        def start_gather(off, g, buf, sem):
            idx = sbuf.at[pl.ds(off + g * BLK, BLK)]

            @pl.when(c == 0)
            def _():
                pltpu.async_copy(z_hbm0.at[idx], buf, sem)

            @pl.when(c == 1)
            def _():
                pltpu.async_copy(z_hbm1.at[idx], buf, sem)

        start_idx(0, 0)

        def phase_b(b, _):
            off = lax.rem(b, 2) * SUP
            drain_idx()

            @pl.when(b + 1 < N_SUP)
            def _():
                start_idx(b + 1, SUP - off)

            start_gather(off, 0, bufs[0], gsems[0])
            start_gather(off, 1, bufs[1], gsems[1])

            def vec(j, _):
                sl = pl.ds(off + j * 16, 16)
                sidx = sbuf[sl]
                didx = dbuf[sl]
                e = jnp.maximum(plsc.load_gather(s1_v, [sidx])
                                + plsc.load_gather(s2_v, [didx]), 0.0)
                ee = jnp.exp(e - plsc.load_gather(m_v, [didx]))
                plsc.addupdate_scatter(den_v, [didx], ee)
                ee_v[pl.ds(j * 16, 16)] = ee
                return 0

            lax.fori_loop(0, SUP // 16, vec, 0)

            for g in range(G_PER_SUP):
                p = g % 3
                drain(bufs[p], gsems[p])
                buf = bufs[p]

                def scale(rr, _):
                    for dr in range(4):
                        r = rr * 4 + dr
                        w = plsc.load_gather(
                            ee_v, [jnp.full((16,), g * BLK + r, jnp.int32)])
                        for kk in range(HALF // 16):
                            sl = pl.ds(kk * 16, 16)
                            buf[r, sl] = buf[r, sl] * w
                    return 0

                lax.fori_loop(0, BLK // 4, scale, 0)

                pltpu.async_copy(
                    buf, h_sh.at[dbuf.at[pl.ds(off + g * BLK, BLK)]],
                    ssems[p], add=True)
                if g + 2 < G_PER_SUP:
                    pn = (g + 2) % 3
                    if g >= 1:
                        drain(bufs[pn], ssems[pn])
                    start_gather(off, g + 2, bufs[pn], gsems[pn])
            for gg in range(G_PER_SUP - 3, G_PER_SUP):
                drain(bufs[gg % 3], ssems[gg % 3])
            return 0

        with scope("phaseB"):
            lax.fori_loop(0, N_SUP, phase_b, 0)

        with scope("combineD"):
            combine(den_v, jnp.add)
        pltpu.sync_copy(acc_v, den_out.at[c, 0, pl.ds(s * SLICE, SLICE)])

        plsc.subcore_barrier()
        pltpu.sync_copy(h_sh.at[pl.ds(s * ROWS_T, ROWS_T)],
                        u_out.at[c, pl.ds(s * ROWS_T, ROWS_T)])

    return k(z0, z1, s1, s2, src, dst)


def kernel(feature, edge_index, fc_weight, attn_weight):
    src = edge_index[0].astype(jnp.int32)
    dst = edge_index[1].astype(jnp.int32)
    attn2 = jnp.concatenate([attn_weight[:DIM], attn_weight[DIM:]], axis=1)

    z0, z1, s = _tc_front(feature, fc_weight, attn2)
    s1 = s[:, 0]
    s2 = s[:, 1]

    u, den = _sc_gat(z0, z1, s1, s2, src, dst)
    den0 = den[0, 0, :N_NODES].reshape(N_NODES, 1)
    return _tc_norm(u[0, :N_NODES], u[1, :N_NODES], den0)


# speedup vs baseline: 1.0699x; 1.0699x over previous
import functools

import jax
import jax.numpy as jnp
from jax import lax
from jax.experimental import pallas as pl
from jax.experimental.pallas import tpu as pltpu
from jax.experimental.pallas import tpu_sc as plsc

N_NODES = 10000
N_EDGES = 320000
DIM = 128
HALF = DIM // 2
NPAD = 10240
E_TILE = N_EDGES // 16
BLK = 80
SUP = 800
N_SUP = E_TILE // SUP
G_PER_SUP = SUP // BLK
SLICE = NPAD // 16
ROWS_T = NPAD // 16


def _tc_front_body(f_ref, w_ref, a_ref, z0_ref, z1_ref, s_ref):
    z = jnp.dot(f_ref[...], w_ref[...], preferred_element_type=jnp.float32)
    z0_ref[...] = z[:, :HALF]
    z1_ref[...] = z[:, HALF:]
    s_ref[...] = jnp.dot(z, a_ref[...], preferred_element_type=jnp.float32)


def _tc_front(feature, fc_weight, attn2):
    blk = 2000
    grid = N_NODES // blk
    return pl.pallas_call(
        _tc_front_body,
        grid=(grid,),
        in_specs=[
            pl.BlockSpec((blk, DIM), lambda i: (i, 0)),
            pl.BlockSpec((DIM, DIM), lambda i: (0, 0)),
            pl.BlockSpec((DIM, 2), lambda i: (0, 0)),
        ],
        out_specs=[
            pl.BlockSpec((blk, HALF), lambda i: (i, 0)),
            pl.BlockSpec((blk, HALF), lambda i: (i, 0)),
            pl.BlockSpec((blk, 2), lambda i: (i, 0)),
        ],
        out_shape=[
            jax.ShapeDtypeStruct((N_NODES, HALF), jnp.float32),
            jax.ShapeDtypeStruct((N_NODES, HALF), jnp.float32),
            jax.ShapeDtypeStruct((N_NODES, 2), jnp.float32),
        ],
    )(feature, fc_weight, attn2)


def _sc_gat(z0, z1, s1, s2, src, dst):
    mesh = plsc.VectorSubcoreMesh(core_axis_name="c", subcore_axis_name="s")

    @functools.partial(
        pl.kernel,
        mesh=mesh,
        compiler_params=pltpu.CompilerParams(needs_layout_passes=False, use_tc_tiling_on_sc=False),
        out_type=jax.ShapeDtypeStruct((NPAD, DIM), jnp.float32),
        scratch_types=[
            pltpu.VMEM((N_NODES,), jnp.float32),
            pltpu.VMEM((N_NODES,), jnp.float32),
            pltpu.VMEM((NPAD,), jnp.float32),
            pltpu.VMEM((NPAD,), jnp.float32),
            pltpu.VMEM((2 * SUP,), jnp.int32),
            pltpu.VMEM((2 * SUP,), jnp.int32),
            pltpu.VMEM((SUP,), jnp.float32),
            pltpu.VMEM((BLK, HALF), jnp.float32),
            pltpu.VMEM((BLK, HALF), jnp.float32),
            pltpu.VMEM((BLK, HALF), jnp.float32),
            pltpu.VMEM((3, BLK), jnp.int32),
            pltpu.VMEM((SLICE,), jnp.float32),
            pltpu.VMEM((16, SLICE), jnp.float32),
            pltpu.VMEM_SHARED((16, 1, NPAD), jnp.float32),
            pltpu.VMEM_SHARED((NPAD,), jnp.float32),
            pltpu.VMEM_SHARED((NPAD, HALF), jnp.float32),
            pltpu.SemaphoreType.DMA,
            pltpu.SemaphoreType.DMA,
            pltpu.SemaphoreType.DMA,
            pltpu.SemaphoreType.DMA,
            pltpu.SemaphoreType.DMA,
            pltpu.SemaphoreType.DMA,
            pltpu.SemaphoreType.DMA,
            pltpu.SemaphoreType.DMA,
            pltpu.SemaphoreType.DMA,
        ],
    )
    def k(z_hbm0, z_hbm1, s1_hbm, s2_hbm, src_hbm, dst_hbm, u_out,
          s1_v, s2_v, m_v, den_v, sbuf, dbuf, ee_v, zbuf0, zbuf1, zbuf2,
          idx_d, acc_v, comb2, red_sh, mfull_sh, h_sh,
          gsem0, gsem1, gsem2, ssem0, ssem1, ssem2, isem_s, isem_d, zsem):
        c = lax.axis_index("c")
        s = lax.axis_index("s")
        ebase = s * E_TILE

        scope = jax.named_scope
        pltpu.async_copy(s1_hbm, s1_v, isem_s)
        pltpu.async_copy(s2_hbm, s2_v, isem_d)

        zf = jnp.zeros((16,), jnp.float32)

        def zinit(i, _):
            m_v[pl.ds(i * 16, 16)] = zf
            den_v[pl.ds(i * 16, 16)] = zf
            return 0

        lax.fori_loop(0, NPAD // 16, zinit, 0)

        def zrow(r, _):
            for kk in range(HALF // 16):
                zbuf0[r, pl.ds(kk * 16, 16)] = zf
            return 0

        lax.fori_loop(0, BLK, zrow, 0)

        for g in range(ROWS_T // BLK):
            pltpu.async_copy(zbuf0, h_sh.at[pl.ds(s * ROWS_T + g * BLK, BLK)],
                             zsem)

        pltpu.make_async_copy(s1_hbm, s1_v, isem_s).wait()
        pltpu.make_async_copy(s2_hbm, s2_v, isem_d).wait()

        def start_idx(b, off):
            base = ebase + b * SUP
            pltpu.async_copy(src_hbm.at[pl.ds(base, SUP)],
                             sbuf.at[pl.ds(off, SUP)], isem_s)
            pltpu.async_copy(dst_hbm.at[pl.ds(base, SUP)],
                             dbuf.at[pl.ds(off, SUP)], isem_d)

        def drain_idx():
            pltpu.make_async_copy(src_hbm.at[pl.ds(0, SUP)],
                                  sbuf.at[pl.ds(0, SUP)], isem_s).wait()
            pltpu.make_async_copy(dst_hbm.at[pl.ds(0, SUP)],
                                  dbuf.at[pl.ds(0, SUP)], isem_d).wait()

        start_idx(0, 0)

        def scatter_max(didx, e):
            def cond(active):
                return active > 0

            def body(active):
                cur = plsc.load_gather(m_v, [didx])
                pend = cur < e
                plsc.store_scatter(m_v, [didx], jnp.maximum(cur, e), mask=pend)
                chk = plsc.load_gather(m_v, [didx])
                return jnp.any(chk < e).astype(jnp.int32)

            lax.while_loop(cond, body, jnp.int32(1))

        def phase_a(b, _):
            off = lax.rem(b, 2) * SUP
            drain_idx()

            @pl.when(b + 1 < N_SUP)
            def _():
                start_idx(b + 1, SUP - off)

            def vec(j, _):
                sl = pl.ds(off + j * 16, 16)
                sidx = sbuf[sl]
                didx = dbuf[sl]
                e = jnp.maximum(plsc.load_gather(s1_v, [sidx])
                                + plsc.load_gather(s2_v, [didx]), 0.0)
                scatter_max(didx, e)
                return 0

            lax.fori_loop(0, SUP // 16, vec, 0)
            return 0

        with scope("phaseA"):
            lax.fori_loop(0, N_SUP, phase_a, 0)

        for g in range(ROWS_T // BLK):
            pltpu.make_async_copy(zbuf0, h_sh.at[pl.ds(0, BLK)], zsem).wait()

        def combine(local_v, op):
            plsc.subcore_barrier()
            pltpu.sync_copy(local_v, red_sh.at[s, 0])
            plsc.subcore_barrier()
            pltpu.sync_copy(red_sh.at[:, 0, pl.ds(s * SLICE, SLICE)], comb2)

            def red(j, _):
                sl = pl.ds(j * 16, 16)
                a = comb2[0, sl]
                for i in range(1, 16):
                    a = op(a, comb2[i, sl])
                acc_v[sl] = a
                return 0

            lax.fori_loop(0, SLICE // 16, red, 0)

        with scope("combineM"):
            combine(m_v, jnp.maximum)
            pltpu.sync_copy(acc_v, mfull_sh.at[pl.ds(s * SLICE, SLICE)])
        plsc.subcore_barrier()
        pltpu.sync_copy(mfull_sh, m_v)

        bufs = (zbuf0, zbuf1, zbuf2)
        gsems = (gsem0, gsem1, gsem2)
        ssems = (ssem0, ssem1, ssem2)

        def drain(buf, sem):
            pltpu.make_async_copy(z_hbm0.at[pl.ds(0, BLK)], buf, sem).wait()

        def start_gather(off, g, buf, sem):
            idx = sbuf.at[pl.ds(off + g * BLK, BLK)]

            @pl.when(c == 0)
            def _():
                pltpu.async_copy(z_hbm0.at[idx], buf, sem)

            @pl.when(c == 1)
            def _():
                pltpu.async_copy(z_hbm1.at[idx], buf, sem)

        start_idx(0, 0)

        def phase_b(b, _):
            off = lax.rem(b, 2) * SUP
            drain_idx()

            @pl.when(b + 1 < N_SUP)
            def _():
                start_idx(b + 1, SUP - off)

            start_gather(off, 0, bufs[0], gsems[0])
            start_gather(off, 1, bufs[1], gsems[1])

            def vec(j, _):
                sl = pl.ds(off + j * 16, 16)
                sidx = sbuf[sl]
                didx = dbuf[sl]
                e = jnp.maximum(plsc.load_gather(s1_v, [sidx])
                                + plsc.load_gather(s2_v, [didx]), 0.0)
                ee = jnp.exp(e - plsc.load_gather(m_v, [didx]))
                plsc.addupdate_scatter(den_v, [didx], ee)
                ee_v[pl.ds(j * 16, 16)] = ee
                return 0

            lax.fori_loop(0, SUP // 16, vec, 0)

            for g in range(G_PER_SUP):
                p = g % 3
                drain(bufs[p], gsems[p])
                buf = bufs[p]

                def scale(rr, _):
                    for dr in range(4):
                        r = rr * 4 + dr
                        w = plsc.load_gather(
                            ee_v, [jnp.full((16,), g * BLK + r, jnp.int32)])
                        for kk in range(HALF // 16):
                            sl = pl.ds(kk * 16, 16)
                            buf[r, sl] = buf[r, sl] * w
                    return 0

                lax.fori_loop(0, BLK // 4, scale, 0)

                pltpu.async_copy(
                    buf, h_sh.at[dbuf.at[pl.ds(off + g * BLK, BLK)]],
                    ssems[p], add=True)
                if g + 2 < G_PER_SUP:
                    pn = (g + 2) % 3
                    if g >= 1:
                        drain(bufs[pn], ssems[pn])
                    start_gather(off, g + 2, bufs[pn], gsems[pn])
            for gg in range(G_PER_SUP - 3, G_PER_SUP):
                drain(bufs[gg % 3], ssems[gg % 3])
            return 0

        with scope("phaseB"):
            lax.fori_loop(0, N_SUP, phase_b, 0)

        with scope("combineD"):
            combine(den_v, jnp.add)

        def dinv(j, _):
            sl = pl.ds(j * 16, 16)
            d = acc_v[sl]
            acc_v[sl] = 1.0 / jnp.where(d == 0.0, 1.0, d)
            return 0

        lax.fori_loop(0, SLICE // 16, dinv, 0)

        plsc.subcore_barrier()
        for g in range(ROWS_T // BLK):
            row0 = s * ROWS_T + g * BLK
            pltpu.sync_copy(h_sh.at[pl.ds(row0, BLK)], zbuf0)

            def nsc(rr, _):
                for dr in range(4):
                    r = rr * 4 + dr
                    w = plsc.load_gather(
                        acc_v, [jnp.full((16,), g * BLK + r, jnp.int32)])
                    for kk in range(HALF // 16):
                        sl = pl.ds(kk * 16, 16)
                        zbuf0[r, sl] = zbuf0[r, sl] * w
                return 0

            lax.fori_loop(0, BLK // 4, nsc, 0)
            pltpu.sync_copy(
                zbuf0,
                u_out.at[pl.ds(row0, BLK), pl.ds(c * HALF, HALF)])

    return k(z0, z1, s1, s2, src, dst)


def kernel(feature, edge_index, fc_weight, attn_weight):
    src = edge_index[0].astype(jnp.int32)
    dst = edge_index[1].astype(jnp.int32)
    attn2 = jnp.concatenate([attn_weight[:DIM], attn_weight[DIM:]], axis=1)

    z0, z1, s = _tc_front(feature, fc_weight, attn2)
    s1 = s[:, 0]
    s2 = s[:, 1]

    u = _sc_gat(z0, z1, s1, s2, src, dst)
    return u[:N_NODES]
